# baseline (speedup 1.0000x reference)
import jax
import jax.numpy as jnp
from jax.experimental import pallas as pl
from jax.experimental.pallas import tpu as pltpu


def _smem_spec():
    return pl.BlockSpec(memory_space=pltpu.MemorySpace.SMEM)


def _resident(shape):
    return pl.BlockSpec(shape, lambda i: (0, 0))


def _edge_kernel(eeps_ref, xs_ref, xd_ref, e_ref,
                 w1_ref, b1_ref, w2_ref, b2_ref, w11_ref, b11_ref,
                 e_new_ref, h1a_ref):
    xs = xs_ref[...]
    xd = xd_ref[...]
    e_old = e_ref[...]

    lhs1 = jnp.concatenate([xs, xd, e_old.astype(jnp.bfloat16)], axis=1)
    h = jnp.dot(lhs1, w1_ref[...], preferred_element_type=jnp.float32) + b1_ref[...]
    h = jnp.maximum(h, 0.0)
    e_upd = jnp.dot(h.astype(jnp.bfloat16), w2_ref[...],
                    preferred_element_type=jnp.float32) + b2_ref[...]
    e_new = (1.0 + eeps_ref[0, 0]) * e_old + e_upd
    e_new_ref[...] = e_new

    lhs2 = jnp.concatenate([xs, e_new.astype(jnp.bfloat16)], axis=1)
    h1 = jnp.dot(lhs2, w11_ref[...], preferred_element_type=jnp.float32) + b11_ref[...]
    h1a_ref[:, :256] = jnp.maximum(h1, 0.0).astype(jnp.bfloat16)
    h1a_ref[:, 256:] = jnp.ones_like(h1a_ref[:, 256:])


def _node_kernel(neps_ref, x_ref, s_ref,
                 w21x_ref, wc_ref, bc_ref, b21_ref, w22_ref, b22_ref,
                 x_new_ref):
    x_old = x_ref[...]
    s = s_ref[...].astype(jnp.float32)
    cnt = s[:, 256:]
    mean_h1 = s[:, :256] / jnp.maximum(cnt, 1.0)

    h2 = (jnp.dot(x_old.astype(jnp.bfloat16), w21x_ref[...],
                  preferred_element_type=jnp.float32)
          + jnp.dot(mean_h1.astype(jnp.bfloat16), wc_ref[...],
                    preferred_element_type=jnp.float32)
          + jnp.where(cnt > 0.0, bc_ref[...], 0.0)
          + b21_ref[...])
    h2 = jnp.maximum(h2, 0.0)
    x_upd = jnp.dot(h2.astype(jnp.bfloat16), w22_ref[...],
                    preferred_element_type=jnp.float32) + b22_ref[...]
    x_new_ref[...] = (1.0 + neps_ref[0, 0]) * x_old + x_upd


def kernel(x, edge_index, edge_attr, edge_eps, node_eps,
           e_w1_xs, e_w1_xd, e_w1_e, e_b1, e_w2, e_b2,
           n_w11_x, n_w11_e, n_b11, n_w12, n_b12,
           n_w21_x, n_w21_a, n_b21, n_w22, n_b22):
    N, Fx = x.shape
    E, Fe = edge_attr.shape
    H = n_w12.shape[1]
    src, dst = edge_index[0], edge_index[1]

    bf16 = jnp.bfloat16
    xb = x.astype(bf16)

    w1 = jnp.concatenate([e_w1_xs, e_w1_xd, e_w1_e], axis=0).astype(bf16)
    w11 = jnp.concatenate([n_w11_x, n_w11_e], axis=0).astype(bf16)
    w2b = e_w2.astype(bf16)
    wc = jnp.dot(n_w12, n_w21_a).astype(bf16)
    bc = jnp.dot(n_b12, n_w21_a)

    cparams = pltpu.CompilerParams(
        dimension_semantics=("parallel",),
        vmem_limit_bytes=64 * 1024 * 1024,
    )

    TE = 2048
    NC = 4
    CE = E // NC
    e_new_parts = []
    sum_aug = jnp.zeros((N, H + 1), bf16)
    for c in range(NC):
        src_c = src[c * CE:(c + 1) * CE]
        dst_c = dst[c * CE:(c + 1) * CE]
        x_src = jnp.take(xb, src_c, axis=0)
        x_dst = jnp.take(xb, dst_c, axis=0)
        e_new_c, h1a_c = pl.pallas_call(
            _edge_kernel,
            out_shape=(jax.ShapeDtypeStruct((CE, Fe), jnp.float32),
                       jax.ShapeDtypeStruct((CE, H + 1), bf16)),
            grid=(pl.cdiv(CE, TE),),
            in_specs=[
                _smem_spec(),
                pl.BlockSpec((TE, Fx), lambda i: (i, 0)),
                pl.BlockSpec((TE, Fx), lambda i: (i, 0)),
                pl.BlockSpec((TE, Fe), lambda i: (i, 0)),
                _resident(w1.shape), _resident(e_b1.shape),
                _resident(e_w2.shape), _resident(e_b2.shape),
                _resident(w11.shape), _resident(n_b11.shape),
            ],
            out_specs=(pl.BlockSpec((TE, Fe), lambda i: (i, 0)),
                       pl.BlockSpec((TE, H + 1), lambda i: (i, 0))),
            compiler_params=cparams,
        )(edge_eps, x_src, x_dst, edge_attr[c * CE:(c + 1) * CE],
          w1, e_b1, w2b, e_b2, w11, n_b11)
        e_new_parts.append(e_new_c)
        sum_aug = sum_aug.at[dst_c].add(h1a_c)

    e_new = jnp.concatenate(e_new_parts, axis=0)

    TN = 2048
    x_new = pl.pallas_call(
        _node_kernel,
        out_shape=jax.ShapeDtypeStruct((N, Fx), jnp.float32),
        grid=(pl.cdiv(N, TN),),
        in_specs=[
            _smem_spec(),
            pl.BlockSpec((TN, Fx), lambda i: (i, 0)),
            pl.BlockSpec((TN, H + 1), lambda i: (i, 0)),
            _resident((Fx, H)), _resident(wc.shape), _resident(bc.shape),
            _resident(n_b21.shape), _resident((H, Fx)), _resident(n_b22.shape),
        ],
        out_specs=pl.BlockSpec((TN, Fx), lambda i: (i, 0)),
        compiler_params=cparams,
    )(node_eps, x, sum_aug,
      n_w21_x.astype(bf16), wc, bc, n_b21, n_w22.astype(bf16), n_b22)

    return x_new, e_new


# speedup vs baseline: 1.7137x; 1.7137x over previous
import jax
import jax.numpy as jnp
from jax.experimental import pallas as pl
from jax.experimental.pallas import tpu as pltpu


def _smem_spec():
    return pl.BlockSpec(memory_space=pltpu.MemorySpace.SMEM)


def _resident(shape):
    return pl.BlockSpec(shape, lambda i: (0, 0))


def _edge_kernel(eeps_ref, xs_ref, xd_ref, e_ref,
                 w1_ref, b1_ref, w2_ref, b2_ref, w11_ref, b11_ref,
                 e_new_ref, h1a_ref):
    xs = xs_ref[...]
    xd = xd_ref[...]
    e_old = e_ref[...]

    lhs1 = jnp.concatenate([xs, xd, e_old.astype(jnp.bfloat16)], axis=1)
    h = jnp.dot(lhs1, w1_ref[...], preferred_element_type=jnp.float32) + b1_ref[...]
    h = jnp.maximum(h, 0.0)
    e_upd = jnp.dot(h.astype(jnp.bfloat16), w2_ref[...],
                    preferred_element_type=jnp.float32) + b2_ref[...]
    e_new = (1.0 + eeps_ref[0, 0]) * e_old + e_upd
    e_new_ref[...] = e_new

    lhs2 = jnp.concatenate([xs, e_new.astype(jnp.bfloat16)], axis=1)
    h1 = jnp.dot(lhs2, w11_ref[...], preferred_element_type=jnp.float32) + b11_ref[...]
    h1a_ref[:, :256] = jnp.maximum(h1, 0.0)
    h1a_ref[:, 256:] = jnp.ones_like(h1a_ref[:, 256:])


def _node_kernel(neps_ref, x_ref, s_ref,
                 w21x_ref, wc_ref, bc_ref, b21_ref, w22_ref, b22_ref,
                 x_new_ref):
    x_old = x_ref[...]
    s = s_ref[...].astype(jnp.float32)
    cnt = s[:, 256:]
    mean_h1 = s[:, :256] / jnp.maximum(cnt, 1.0)

    h2 = (jnp.dot(x_old.astype(jnp.bfloat16), w21x_ref[...],
                  preferred_element_type=jnp.float32)
          + jnp.dot(mean_h1.astype(jnp.bfloat16), wc_ref[...],
                    preferred_element_type=jnp.float32)
          + jnp.where(cnt > 0.0, bc_ref[...], 0.0)
          + b21_ref[...])
    h2 = jnp.maximum(h2, 0.0)
    x_upd = jnp.dot(h2.astype(jnp.bfloat16), w22_ref[...],
                    preferred_element_type=jnp.float32) + b22_ref[...]
    x_new_ref[...] = (1.0 + neps_ref[0, 0]) * x_old + x_upd


def kernel(x, edge_index, edge_attr, edge_eps, node_eps,
           e_w1_xs, e_w1_xd, e_w1_e, e_b1, e_w2, e_b2,
           n_w11_x, n_w11_e, n_b11, n_w12, n_b12,
           n_w21_x, n_w21_a, n_b21, n_w22, n_b22):
    N, Fx = x.shape
    E, Fe = edge_attr.shape
    H = n_w12.shape[1]
    src, dst = edge_index[0], edge_index[1]

    bf16 = jnp.bfloat16
    xb = x.astype(bf16)

    w1 = jnp.concatenate([e_w1_xs, e_w1_xd, e_w1_e], axis=0).astype(bf16)
    w11 = jnp.concatenate([n_w11_x, n_w11_e], axis=0).astype(bf16)
    w2b = e_w2.astype(bf16)
    wc = jnp.dot(n_w12, n_w21_a).astype(bf16)
    bc = jnp.dot(n_b12, n_w21_a)

    cparams = pltpu.CompilerParams(
        dimension_semantics=("parallel",),
        vmem_limit_bytes=64 * 1024 * 1024,
    )

    TE = 2048
    NC = 4
    CE = E // NC
    e_new_parts = []
    sum_aug = None
    for c in range(NC):
        src_c = src[c * CE:(c + 1) * CE]
        dst_c = dst[c * CE:(c + 1) * CE]
        x_src = jnp.take(xb, src_c, axis=0)
        x_dst = jnp.take(xb, dst_c, axis=0)
        e_new_c, h1a_c = pl.pallas_call(
            _edge_kernel,
            out_shape=(jax.ShapeDtypeStruct((CE, Fe), jnp.float32),
                       jax.ShapeDtypeStruct((CE, H + 1), jnp.float32)),
            grid=(pl.cdiv(CE, TE),),
            in_specs=[
                _smem_spec(),
                pl.BlockSpec((TE, Fx), lambda i: (i, 0)),
                pl.BlockSpec((TE, Fx), lambda i: (i, 0)),
                pl.BlockSpec((TE, Fe), lambda i: (i, 0)),
                _resident(w1.shape), _resident(e_b1.shape),
                _resident(e_w2.shape), _resident(e_b2.shape),
                _resident(w11.shape), _resident(n_b11.shape),
            ],
            out_specs=(pl.BlockSpec((TE, Fe), lambda i: (i, 0)),
                       pl.BlockSpec((TE, H + 1), lambda i: (i, 0))),
            compiler_params=cparams,
        )(edge_eps, x_src, x_dst, edge_attr[c * CE:(c + 1) * CE],
          w1, e_b1, w2b, e_b2, w11, n_b11)
        e_new_parts.append(e_new_c)
        s_c = jax.ops.segment_sum(h1a_c, dst_c, num_segments=N)
        sum_aug = s_c if sum_aug is None else sum_aug + s_c

    e_new = jnp.concatenate(e_new_parts, axis=0)

    TN = 2048
    x_new = pl.pallas_call(
        _node_kernel,
        out_shape=jax.ShapeDtypeStruct((N, Fx), jnp.float32),
        grid=(pl.cdiv(N, TN),),
        in_specs=[
            _smem_spec(),
            pl.BlockSpec((TN, Fx), lambda i: (i, 0)),
            pl.BlockSpec((TN, H + 1), lambda i: (i, 0)),
            _resident((Fx, H)), _resident(wc.shape), _resident(bc.shape),
            _resident(n_b21.shape), _resident((H, Fx)), _resident(n_b22.shape),
        ],
        out_specs=pl.BlockSpec((TN, Fx), lambda i: (i, 0)),
        compiler_params=cparams,
    )(node_eps, x, sum_aug,
      n_w21_x.astype(bf16), wc, bc, n_b21, n_w22.astype(bf16), n_b22)

    return x_new, e_new
